# baseline (speedup 1.0000x reference)
import functools

import jax
import jax.numpy as jnp
from jax import lax
from jax.experimental import pallas as pl
from jax.experimental.pallas import tpu as pltpu
from jax.experimental.pallas import tpu_sc as plsc

PC_MIN_X, PC_MIN_Y, PC_MIN_Z = -50.0, -50.0, -5.0
NX, NY, NZ, B = 100, 100, 8, 4
SEG_PER_B = NX * NY * NZ
NUM_SEG = B * SEG_PER_B
HALF = NUM_SEG // 2
PAD_ROWS = 1024
ACC_ROWS = HALF + PAD_ROWS
N_POINTS = 1600000
NUM_CORES, NUM_SUBCORES, LANES = 2, 16, 16
PTS_PER_TILE = N_POINTS // NUM_SUBCORES
CHUNK = 4000
N_CHUNKS = PTS_PER_TILE // CHUNK
GROUPS = CHUNK // LANES
CHUNK2 = 2000
N_CHUNKS2 = PTS_PER_TILE // CHUNK2
GROUPS2 = CHUNK2 // LANES

N_PAD = 314 * 1024
BLK_R, BLK_C = 8, 128
N_ROWS = N_PAD // BLK_C
SWEEPS = 6

_MESH = plsc.VectorSubcoreMesh(
    core_axis_name="c", subcore_axis_name="s",
    num_cores=NUM_CORES, num_subcores=NUM_SUBCORES)


def _seg_index(bval, x, y, z):
    i32 = jnp.int32
    cx = (x - jnp.float32(PC_MIN_X)).astype(i32)
    cy = (y - jnp.float32(PC_MIN_Y)).astype(i32)
    cz = (z - jnp.float32(PC_MIN_Z)).astype(i32)
    zero = jnp.zeros((LANES,), i32)
    cx = jnp.minimum(jnp.maximum(cx, zero), jnp.full((LANES,), NX - 1, i32))
    cy = jnp.minimum(jnp.maximum(cy, zero), jnp.full((LANES,), NY - 1, i32))
    cz = jnp.minimum(jnp.maximum(cz, zero), jnp.full((LANES,), NZ - 1, i32))
    bi = bval.astype(i32)
    return (cx * NY + cy) * NZ + cz + bi * SEG_PER_B


def _sc_pass1_body(points_flat, zeros_acc, out4, seg_out,
                   acc4, sidx4_buf, seg_buf, chunk_flat, upd4):
    c = lax.axis_index("c")
    s = lax.axis_index("s")
    lo = c * HALF

    @pl.when(s == 0)
    def _init():
        for k in range(4):
            pltpu.sync_copy(zeros_acc, acc4.at[pl.ds(k * ACC_ROWS, ACC_ROWS)])

    def _ones_init(g, _):
        upd4[pl.ds(g * LANES, LANES)] = jnp.ones((LANES,), jnp.float32)
        return 0

    lax.fori_loop(0, GROUPS, _ones_init, 0)
    plsc.subcore_barrier()

    tile_base = s * PTS_PER_TILE
    lanes = lax.iota(jnp.int32, LANES)

    def _chunk(i, _):
        base = tile_base + i * CHUNK
        pltpu.sync_copy(points_flat.at[pl.ds(base * 4, CHUNK * 4)], chunk_flat)

        def _group(g, _):
            rows = g * LANES + lanes
            r4 = rows * 4
            bval = plsc.load_gather(chunk_flat, [r4])
            x = plsc.load_gather(chunk_flat, [r4 + 1])
            y = plsc.load_gather(chunk_flat, [r4 + 2])
            z = plsc.load_gather(chunk_flat, [r4 + 3])
            seg = _seg_index(bval, x, y, z)
            local = seg - lo
            in_half = (local >= 0) & (local < HALF)
            pad = HALF + ((i * CHUNK + rows) & (PAD_ROWS - 1))
            sidx = jnp.where(in_half, local, pad)
            sl = pl.ds(g * LANES, LANES)
            seg_buf[sl] = seg
            for k in range(4):
                sidx4_buf[pl.ds(k * CHUNK + g * LANES, LANES)] = (
                    sidx + k * ACC_ROWS)
            upd4[pl.ds(CHUNK + g * LANES, LANES)] = x
            upd4[pl.ds(2 * CHUNK + g * LANES, LANES)] = y
            upd4[pl.ds(3 * CHUNK + g * LANES, LANES)] = z
            return 0

        lax.fori_loop(0, GROUPS, _group, 0)
        pltpu.sync_copy(upd4, acc4.at[sidx4_buf], add=True)

        @pl.when(c == 0)
        def _save_seg():
            pltpu.sync_copy(seg_buf, seg_out.at[pl.ds(base, CHUNK)])

        return 0

    lax.fori_loop(0, N_CHUNKS, _chunk, 0)
    plsc.subcore_barrier()

    @pl.when(s == 0)
    def _flush():
        for k in range(4):
            pltpu.sync_copy(acc4.at[pl.ds(k * ACC_ROWS, HALF)],
                            out4.at[pl.ds(k * N_PAD + c * HALF, HALF)])


def _sc_pass1(points_flat, zeros_acc):
    return pl.kernel(
        _sc_pass1_body,
        out_type=[
            jax.ShapeDtypeStruct((4 * N_PAD,), jnp.float32),
            jax.ShapeDtypeStruct((N_POINTS,), jnp.int32),
        ],
        mesh=_MESH,
        compiler_params=pltpu.CompilerParams(needs_layout_passes=False),
        scratch_types=(
            [pltpu.VMEM_SHARED((4 * ACC_ROWS,), jnp.float32)]
            + [pltpu.VMEM((4 * CHUNK,), jnp.int32)]
            + [pltpu.VMEM((CHUNK,), jnp.int32)]
            + [pltpu.VMEM((CHUNK * 4,), jnp.float32)]
            + [pltpu.VMEM((4 * CHUNK,), jnp.float32)]
        ),
    )(points_flat, zeros_acc)


def _sc_pass2_body(points_flat, seg_in, mxs, mys, mzs, zeros_acc, out6,
                   acc6, mean3, sidx6_buf, gidx3_buf, chunk_flat, pbuf):
    c = lax.axis_index("c")
    s = lax.axis_index("s")
    lo = c * HALF

    @pl.when(s == 0)
    def _init():
        for k in range(6):
            pltpu.sync_copy(zeros_acc, acc6.at[pl.ds(k * ACC_ROWS, ACC_ROWS)])
        pltpu.sync_copy(mxs.at[pl.ds(lo, HALF)], mean3.at[pl.ds(0, HALF)])
        pltpu.sync_copy(mys.at[pl.ds(lo, HALF)], mean3.at[pl.ds(HALF, HALF)])
        pltpu.sync_copy(mzs.at[pl.ds(lo, HALF)],
                        mean3.at[pl.ds(2 * HALF, HALF)])

    plsc.subcore_barrier()

    tile_base = s * PTS_PER_TILE
    lanes = lax.iota(jnp.int32, LANES)

    def _chunk(i, _):
        base = tile_base + i * CHUNK2
        pltpu.sync_copy(points_flat.at[pl.ds(base * 4, CHUNK2 * 4)],
                        chunk_flat)
        pltpu.sync_copy(seg_in.at[pl.ds(base, CHUNK2)],
                        gidx3_buf.at[pl.ds(0, CHUNK2)])

        def _group_idx(g, _):
            sl = pl.ds(g * LANES, LANES)
            rows = g * LANES + lanes
            seg = gidx3_buf[sl]
            local = seg - lo
            in_half = (local >= 0) & (local < HALF)
            pad = HALF + ((i * CHUNK2 + rows) & (PAD_ROWS - 1))
            sidx = jnp.where(in_half, local, pad)
            gidx = jnp.where(in_half, local, 0)
            for k in range(6):
                sidx6_buf[pl.ds(k * CHUNK2 + g * LANES, LANES)] = (
                    sidx + k * ACC_ROWS)
            gidx3_buf[sl] = gidx
            gidx3_buf[pl.ds(CHUNK2 + g * LANES, LANES)] = gidx + HALF
            gidx3_buf[pl.ds(2 * CHUNK2 + g * LANES, LANES)] = gidx + 2 * HALF
            return 0

        lax.fori_loop(0, GROUPS2, _group_idx, 0)
        pltpu.sync_copy(mean3.at[gidx3_buf], pbuf.at[pl.ds(0, 3 * CHUNK2)])

        def _group_prod(g, _):
            rows = g * LANES + lanes
            r4 = rows * 4
            x = plsc.load_gather(chunk_flat, [r4 + 1])
            y = plsc.load_gather(chunk_flat, [r4 + 2])
            z = plsc.load_gather(chunk_flat, [r4 + 3])
            cx = x - pbuf[pl.ds(g * LANES, LANES)]
            cy = y - pbuf[pl.ds(CHUNK2 + g * LANES, LANES)]
            cz = z - pbuf[pl.ds(2 * CHUNK2 + g * LANES, LANES)]
            pbuf[pl.ds(g * LANES, LANES)] = cx * cx
            pbuf[pl.ds(CHUNK2 + g * LANES, LANES)] = cx * cy
            pbuf[pl.ds(2 * CHUNK2 + g * LANES, LANES)] = cx * cz
            pbuf[pl.ds(3 * CHUNK2 + g * LANES, LANES)] = cy * cy
            pbuf[pl.ds(4 * CHUNK2 + g * LANES, LANES)] = cy * cz
            pbuf[pl.ds(5 * CHUNK2 + g * LANES, LANES)] = cz * cz
            return 0

        lax.fori_loop(0, GROUPS2, _group_prod, 0)
        pltpu.sync_copy(pbuf, acc6.at[sidx6_buf], add=True)
        return 0

    lax.fori_loop(0, N_CHUNKS2, _chunk, 0)
    plsc.subcore_barrier()

    @pl.when(s == 0)
    def _flush():
        for k in range(6):
            pltpu.sync_copy(acc6.at[pl.ds(k * ACC_ROWS, HALF)],
                            out6.at[pl.ds(k * N_PAD + c * HALF, HALF)])


def _sc_pass2(points_flat, seg, mxs, mys, mzs, zeros_acc):
    return pl.kernel(
        _sc_pass2_body,
        out_type=[
            jax.ShapeDtypeStruct((6 * N_PAD,), jnp.float32),
        ],
        mesh=_MESH,
        compiler_params=pltpu.CompilerParams(needs_layout_passes=False),
        scratch_types=(
            [pltpu.VMEM_SHARED((6 * ACC_ROWS,), jnp.float32)]
            + [pltpu.VMEM_SHARED((3 * HALF,), jnp.float32)]
            + [pltpu.VMEM((6 * CHUNK2,), jnp.int32)]
            + [pltpu.VMEM((3 * CHUNK2,), jnp.int32)]
            + [pltpu.VMEM((CHUNK2 * 4,), jnp.float32)]
            + [pltpu.VMEM((6 * CHUNK2,), jnp.float32)]
        ),
    )(points_flat, seg, mxs, mys, mzs, zeros_acc)


def _jacobi_rotate(A, V):
    one = jnp.float32(1.0)
    cs = []
    for p, q in ((0, 2), (1, 3)):
        app, aqq, apq = A[p][p], A[q][q], A[p][q]
        tau = (app - aqq) / (jnp.float32(2.0) * apq)
        sgn = jnp.where(tau > 0, one, jnp.float32(-1.0))
        t = sgn / (jnp.abs(tau) + jnp.sqrt(one + tau * tau))
        t = jnp.where(apq == 0.0, jnp.float32(0.0), t)
        cc = one / jnp.sqrt(t * t + one)
        ss = -t * cc
        cs.append((cc, ss))
    An = [[None] * 4 for _ in range(4)]
    for i, (p, q) in enumerate(((0, 2), (1, 3))):
        cc, ss = cs[i]
        for j in range(4):
            An[p][j] = cc * A[p][j] - ss * A[q][j]
            An[q][j] = ss * A[p][j] + cc * A[q][j]
    A2 = [[None] * 4 for _ in range(4)]
    for i, (p, q) in enumerate(((0, 2), (1, 3))):
        cc, ss = cs[i]
        for r in range(4):
            A2[r][p] = cc * An[r][p] - ss * An[r][q]
            A2[r][q] = ss * An[r][p] + cc * An[r][q]
    V2 = [row[:] for row in V]
    for i, (p, q) in enumerate(((0, 2), (1, 3))):
        cc, ss = cs[i]
        for r in range(4):
            V2[r][p] = cc * V[r][p] - ss * V[r][q]
            V2[r][q] = ss * V[r][p] + cc * V[r][q]
    perm = (0, 2, 3, 1)
    A3 = [[A2[perm[r]][perm[j]] for j in range(4)] for r in range(4)]
    V3 = [[V2[r][perm[j]] for j in range(4)] for r in range(4)]
    return A3, V3


def _eigh3_replica(cxx, cxy, cxz, cyy, cyz, czz):
    zero = jnp.zeros_like(cxx)
    one_v = jnp.ones_like(cxx)
    A = [[cxx, cxy, cxz, zero],
         [cxy, cyy, cyz, zero],
         [cxz, cyz, czz, zero],
         [zero, zero, zero, zero]]
    V = [[one_v, zero, zero, zero],
         [zero, one_v, zero, zero],
         [zero, zero, one_v, zero],
         [zero, zero, zero, one_v]]
    for _ in range(SWEEPS * 3):
        A, V = _jacobi_rotate(A, V)
    w = [A[0][0], A[1][1], A[2][2]]
    cols = [[V[r][j] for r in range(3)] for j in range(3)]

    def cswap(wa, ca, wb, cb):
        pred = wa > wb
        wa2 = jnp.where(pred, wb, wa)
        wb2 = jnp.where(pred, wa, wb)
        ca2 = [jnp.where(pred, yy, xx) for xx, yy in zip(ca, cb)]
        cb2 = [jnp.where(pred, xx, yy) for xx, yy in zip(ca, cb)]
        return wa2, ca2, wb2, cb2

    w[0], cols[0], w[1], cols[1] = cswap(w[0], cols[0], w[1], cols[1])
    w[1], cols[1], w[2], cols[2] = cswap(w[1], cols[1], w[2], cols[2])
    w[0], cols[0], w[1], cols[1] = cswap(w[0], cols[0], w[1], cols[1])
    return w, cols


def _means_kernel(cnt_r, sx_r, sy_r, sz_r, mb_r, mx_r, my_r, mz_r):
    i = pl.program_id(0)
    cnt = cnt_r[...]
    denom = jnp.maximum(cnt, jnp.float32(1.0))
    row = lax.broadcasted_iota(jnp.int32, (BLK_R, BLK_C), 0)
    col = lax.broadcasted_iota(jnp.int32, (BLK_R, BLK_C), 1)
    pos = (i * BLK_R + row) * BLK_C + col
    b0 = (pos // SEG_PER_B).astype(jnp.float32)
    mb_r[...] = jnp.where(cnt > 0, b0, jnp.float32(0.0))
    mx_r[...] = sx_r[...] / denom
    my_r[...] = sy_r[...] / denom
    mz_r[...] = sz_r[...] / denom


def _planes_kernel(cnt_r, mx_r, my_r, mz_r,
                   sxx_r, sxy_r, sxz_r, syy_r, syz_r, szz_r,
                   n0_r, n1_r, n2_r, off_r, e0_r, e1_r, e2_r):
    denom = jnp.maximum(cnt_r[...], jnp.float32(1.0))
    cxx = sxx_r[...] / denom + jnp.float32(1e-4)
    cxy = sxy_r[...] / denom
    cxz = sxz_r[...] / denom
    cyy = syy_r[...] / denom + jnp.float32(2e-4)
    cyz = syz_r[...] / denom
    czz = szz_r[...] / denom + jnp.float32(3e-4)
    w, cols = _eigh3_replica(cxx, cxy, cxz, cyy, cyz, czz)
    n0, n1, n2 = cols[0]
    mx, my, mz = mx_r[...], my_r[...], mz_r[...]
    n0_r[...] = n0
    n1_r[...] = n1
    n2_r[...] = n2
    off_r[...] = -((n0 * mx + n1 * my) + n2 * mz)
    e0_r[...] = w[0]
    e1_r[...] = w[1]
    e2_r[...] = w[2]


def _tc_call(body, n_in, n_out, args):
    spec = pl.BlockSpec((BLK_R, BLK_C), lambda i: (i, 0))
    shape = jax.ShapeDtypeStruct((N_ROWS, BLK_C), jnp.float32)
    return pl.pallas_call(
        body,
        grid=(N_ROWS // BLK_R,),
        in_specs=[spec] * n_in,
        out_specs=[spec] * n_out,
        out_shape=[shape] * n_out,
    )(*args)


def _interleave(channels):
    ncol = len(channels)
    n = channels[0].shape[0]
    col = lax.broadcasted_iota(jnp.int32, (n, ncol), 1)
    out = jnp.zeros((n, ncol), jnp.float32)
    for k, ch in enumerate(channels):
        out = jnp.where(col == k, ch[:, None], out)
    return out


@jax.jit
def kernel(point_bxyz):
    points_flat = jnp.zeros((N_POINTS * 4,), jnp.float32) + point_bxyz[0, 0]
    zeros_acc = jnp.zeros((ACC_ROWS,), jnp.float32)
    out4, seg = _sc_pass1(points_flat, zeros_acc)
    ch4 = out4.reshape(4, N_ROWS, BLK_C)

    mb, mx, my, mz = _tc_call(
        _means_kernel, 4, 4, [ch4[0], ch4[1], ch4[2], ch4[3]])

    def _unpad(a):
        return a.reshape(N_PAD)[:NUM_SEG]

    mean_bxyz = _interleave([_unpad(mb), _unpad(mx), _unpad(my), _unpad(mz)])

    (out6,) = _sc_pass2(points_flat, seg, mx.reshape(-1), my.reshape(-1),
                        mz.reshape(-1), zeros_acc)
    ch6 = out6.reshape(6, N_ROWS, BLK_C)

    planes_ch = _tc_call(
        _planes_kernel, 10, 7,
        [ch4[0], mx, my, mz,
         ch6[0], ch6[1], ch6[2], ch6[3], ch6[4], ch6[5]])

    planes = _interleave([_unpad(o) for o in planes_ch])
    return (mean_bxyz, planes)


# speedup vs baseline: 7.0796x; 7.0796x over previous
import functools

import jax
import jax.numpy as jnp
from jax import lax
from jax.experimental import pallas as pl
from jax.experimental.pallas import tpu as pltpu
from jax.experimental.pallas import tpu_sc as plsc

PC_MIN_X, PC_MIN_Y, PC_MIN_Z = -50.0, -50.0, -5.0
NX, NY, NZ, B = 100, 100, 8, 4
SEG_PER_B = NX * NY * NZ
NUM_SEG = B * SEG_PER_B
HALF = NUM_SEG // 2
PAD_ROWS = 1024
ACC_ROWS = HALF + PAD_ROWS
N_POINTS = 1600000
NUM_CORES, NUM_SUBCORES, LANES = 2, 16, 16
PTS_PER_TILE = N_POINTS // NUM_SUBCORES
TILE_PTS = 128
N_TILES = N_POINTS // TILE_PTS
K1 = 25
CHUNK = K1 * TILE_PTS
N_CHUNKS_TOTAL = N_TILES // K1
P1_ITERS = -(-N_CHUNKS_TOTAL // NUM_SUBCORES)
GROUPS = CHUNK // LANES
CHUNK2 = 2000
N_CHUNKS2 = PTS_PER_TILE // CHUNK2
GROUPS2 = CHUNK2 // LANES

N_PAD = 314 * 1024
BLK_R, BLK_C = 8, 128
N_ROWS = N_PAD // BLK_C
SWEEPS = 6

_MESH = plsc.VectorSubcoreMesh(
    core_axis_name="c", subcore_axis_name="s",
    num_cores=NUM_CORES, num_subcores=NUM_SUBCORES)


def _seg_index(bval, x, y, z):
    i32 = jnp.int32
    cx = (x - jnp.float32(PC_MIN_X)).astype(i32)
    cy = (y - jnp.float32(PC_MIN_Y)).astype(i32)
    cz = (z - jnp.float32(PC_MIN_Z)).astype(i32)
    zero = jnp.zeros((LANES,), i32)
    cx = jnp.minimum(jnp.maximum(cx, zero), jnp.full((LANES,), NX - 1, i32))
    cy = jnp.minimum(jnp.maximum(cy, zero), jnp.full((LANES,), NY - 1, i32))
    cz = jnp.minimum(jnp.maximum(cz, zero), jnp.full((LANES,), NZ - 1, i32))
    bi = bval.astype(i32)
    return (cx * NY + cy) * NZ + cz + bi * SEG_PER_B


def _sc_pass1_body(points_lin, zeros_acc, out4, seg_out, xyz_out,
                   acc4, sidx4_buf, seg_buf, chunk_flat, upd4):
    c = lax.axis_index("c")
    s = lax.axis_index("s")
    lo = c * HALF

    @pl.when(s == 0)
    def _init():
        for k in range(4):
            pltpu.sync_copy(zeros_acc, acc4.at[pl.ds(k * ACC_ROWS, ACC_ROWS)])

    def _ones_init(g, _):
        upd4[pl.ds(g * LANES, LANES)] = jnp.ones((LANES,), jnp.float32)
        return 0

    lax.fori_loop(0, GROUPS, _ones_init, 0)
    plsc.subcore_barrier()

    lanes = lax.iota(jnp.int32, LANES)

    def _chunk(j, _):
        chunk_id = j * NUM_SUBCORES + s

        @pl.when(chunk_id < N_CHUNKS_TOTAL)
        def _do():
            base = chunk_id * CHUNK
            pltpu.sync_copy(points_lin.at[pl.ds(base * 4, CHUNK * 4)],
                            chunk_flat)

            def _group(g, _):
                toff = (g // 8) * 512 + (g % 8) * LANES
                bval = chunk_flat[pl.ds(toff, LANES)]
                x = chunk_flat[pl.ds(toff + 128, LANES)]
                y = chunk_flat[pl.ds(toff + 256, LANES)]
                z = chunk_flat[pl.ds(toff + 384, LANES)]
                seg = _seg_index(bval, x, y, z)
                local = seg - lo
                in_half = (local >= 0) & (local < HALF)
                rows = g * LANES + lanes
                pad = HALF + ((chunk_id * CHUNK + rows) & (PAD_ROWS - 1))
                sidx = jnp.where(in_half, local, pad)
                sl = pl.ds(g * LANES, LANES)
                seg_buf[sl] = seg
                for k in range(4):
                    sidx4_buf[pl.ds(k * CHUNK + g * LANES, LANES)] = (
                        sidx + k * ACC_ROWS)
                upd4[pl.ds(CHUNK + g * LANES, LANES)] = x
                upd4[pl.ds(2 * CHUNK + g * LANES, LANES)] = y
                upd4[pl.ds(3 * CHUNK + g * LANES, LANES)] = z
                return 0

            lax.fori_loop(0, GROUPS, _group, 0)
            pltpu.sync_copy(upd4, acc4.at[sidx4_buf], add=True)

            @pl.when(c == 0)
            def _save():
                pltpu.sync_copy(seg_buf, seg_out.at[pl.ds(base, CHUNK)])
                for k in range(3):
                    pltpu.sync_copy(
                        upd4.at[pl.ds((k + 1) * CHUNK, CHUNK)],
                        xyz_out.at[pl.ds(k * N_POINTS + base, CHUNK)])

        return 0

    lax.fori_loop(0, P1_ITERS, _chunk, 0)
    plsc.subcore_barrier()

    @pl.when(s == 0)
    def _flush():
        for k in range(4):
            pltpu.sync_copy(acc4.at[pl.ds(k * ACC_ROWS, HALF)],
                            out4.at[pl.ds(k * N_PAD + c * HALF, HALF)])


def _sc_pass1(points_lin, zeros_acc):
    return pl.kernel(
        _sc_pass1_body,
        out_type=[
            jax.ShapeDtypeStruct((4 * N_PAD,), jnp.float32),
            jax.ShapeDtypeStruct((N_POINTS,), jnp.int32),
            jax.ShapeDtypeStruct((3 * N_POINTS,), jnp.float32),
        ],
        mesh=_MESH,
        compiler_params=pltpu.CompilerParams(needs_layout_passes=False),
        scratch_types=(
            [pltpu.VMEM_SHARED((4 * ACC_ROWS,), jnp.float32)]
            + [pltpu.VMEM((4 * CHUNK,), jnp.int32)]
            + [pltpu.VMEM((CHUNK,), jnp.int32)]
            + [pltpu.VMEM((CHUNK * 4,), jnp.float32)]
            + [pltpu.VMEM((4 * CHUNK,), jnp.float32)]
        ),
    )(points_lin, zeros_acc)


def _sc_pass2_body(xyz_in, seg_in, mxs, mys, mzs, zeros_acc, out6,
                   acc6, mean3, sidx6_buf, gidx3_buf, xyz_buf, pbuf):
    c = lax.axis_index("c")
    s = lax.axis_index("s")
    lo = c * HALF

    @pl.when(s == 0)
    def _init():
        for k in range(6):
            pltpu.sync_copy(zeros_acc, acc6.at[pl.ds(k * ACC_ROWS, ACC_ROWS)])
        pltpu.sync_copy(mxs.at[pl.ds(lo, HALF)], mean3.at[pl.ds(0, HALF)])
        pltpu.sync_copy(mys.at[pl.ds(lo, HALF)], mean3.at[pl.ds(HALF, HALF)])
        pltpu.sync_copy(mzs.at[pl.ds(lo, HALF)],
                        mean3.at[pl.ds(2 * HALF, HALF)])

    plsc.subcore_barrier()

    tile_base = s * PTS_PER_TILE
    lanes = lax.iota(jnp.int32, LANES)

    def _chunk(i, _):
        base = tile_base + i * CHUNK2
        for k in range(3):
            pltpu.sync_copy(xyz_in.at[pl.ds(k * N_POINTS + base, CHUNK2)],
                            xyz_buf.at[pl.ds(k * CHUNK2, CHUNK2)])
        pltpu.sync_copy(seg_in.at[pl.ds(base, CHUNK2)],
                        gidx3_buf.at[pl.ds(0, CHUNK2)])

        def _group_idx(g, _):
            sl = pl.ds(g * LANES, LANES)
            rows = g * LANES + lanes
            seg = gidx3_buf[sl]
            local = seg - lo
            in_half = (local >= 0) & (local < HALF)
            pad = HALF + ((i * CHUNK2 + rows) & (PAD_ROWS - 1))
            sidx = jnp.where(in_half, local, pad)
            gidx = jnp.where(in_half, local, 0)
            for k in range(6):
                sidx6_buf[pl.ds(k * CHUNK2 + g * LANES, LANES)] = (
                    sidx + k * ACC_ROWS)
            gidx3_buf[sl] = gidx
            gidx3_buf[pl.ds(CHUNK2 + g * LANES, LANES)] = gidx + HALF
            gidx3_buf[pl.ds(2 * CHUNK2 + g * LANES, LANES)] = gidx + 2 * HALF
            return 0

        lax.fori_loop(0, GROUPS2, _group_idx, 0)
        pltpu.sync_copy(mean3.at[gidx3_buf], pbuf.at[pl.ds(0, 3 * CHUNK2)])

        def _group_prod(g, _):
            sl = pl.ds(g * LANES, LANES)
            x = xyz_buf[sl]
            y = xyz_buf[pl.ds(CHUNK2 + g * LANES, LANES)]
            z = xyz_buf[pl.ds(2 * CHUNK2 + g * LANES, LANES)]
            cx = x - pbuf[sl]
            cy = y - pbuf[pl.ds(CHUNK2 + g * LANES, LANES)]
            cz = z - pbuf[pl.ds(2 * CHUNK2 + g * LANES, LANES)]
            pbuf[sl] = cx * cx
            pbuf[pl.ds(CHUNK2 + g * LANES, LANES)] = cx * cy
            pbuf[pl.ds(2 * CHUNK2 + g * LANES, LANES)] = cx * cz
            pbuf[pl.ds(3 * CHUNK2 + g * LANES, LANES)] = cy * cy
            pbuf[pl.ds(4 * CHUNK2 + g * LANES, LANES)] = cy * cz
            pbuf[pl.ds(5 * CHUNK2 + g * LANES, LANES)] = cz * cz
            return 0

        lax.fori_loop(0, GROUPS2, _group_prod, 0)
        pltpu.sync_copy(pbuf, acc6.at[sidx6_buf], add=True)
        return 0

    lax.fori_loop(0, N_CHUNKS2, _chunk, 0)
    plsc.subcore_barrier()

    @pl.when(s == 0)
    def _flush():
        for k in range(6):
            pltpu.sync_copy(acc6.at[pl.ds(k * ACC_ROWS, HALF)],
                            out6.at[pl.ds(k * N_PAD + c * HALF, HALF)])


def _sc_pass2(xyz_soa, seg, mxs, mys, mzs, zeros_acc):
    return pl.kernel(
        _sc_pass2_body,
        out_type=[
            jax.ShapeDtypeStruct((6 * N_PAD,), jnp.float32),
        ],
        mesh=_MESH,
        compiler_params=pltpu.CompilerParams(needs_layout_passes=False),
        scratch_types=(
            [pltpu.VMEM_SHARED((6 * ACC_ROWS,), jnp.float32)]
            + [pltpu.VMEM_SHARED((3 * HALF,), jnp.float32)]
            + [pltpu.VMEM((6 * CHUNK2,), jnp.int32)]
            + [pltpu.VMEM((3 * CHUNK2,), jnp.int32)]
            + [pltpu.VMEM((3 * CHUNK2,), jnp.float32)]
            + [pltpu.VMEM((6 * CHUNK2,), jnp.float32)]
        ),
    )(xyz_soa, seg, mxs, mys, mzs, zeros_acc)


def _jacobi_rotate(A, V):
    one = jnp.float32(1.0)
    cs = []
    for p, q in ((0, 2), (1, 3)):
        app, aqq, apq = A[p][p], A[q][q], A[p][q]
        tau = (app - aqq) / (jnp.float32(2.0) * apq)
        sgn = jnp.where(tau > 0, one, jnp.float32(-1.0))
        t = sgn / (jnp.abs(tau) + jnp.sqrt(one + tau * tau))
        t = jnp.where(apq == 0.0, jnp.float32(0.0), t)
        cc = one / jnp.sqrt(t * t + one)
        ss = -t * cc
        cs.append((cc, ss))
    An = [[None] * 4 for _ in range(4)]
    for i, (p, q) in enumerate(((0, 2), (1, 3))):
        cc, ss = cs[i]
        for j in range(4):
            An[p][j] = cc * A[p][j] - ss * A[q][j]
            An[q][j] = ss * A[p][j] + cc * A[q][j]
    A2 = [[None] * 4 for _ in range(4)]
    for i, (p, q) in enumerate(((0, 2), (1, 3))):
        cc, ss = cs[i]
        for r in range(4):
            A2[r][p] = cc * An[r][p] - ss * An[r][q]
            A2[r][q] = ss * An[r][p] + cc * An[r][q]
    V2 = [row[:] for row in V]
    for i, (p, q) in enumerate(((0, 2), (1, 3))):
        cc, ss = cs[i]
        for r in range(4):
            V2[r][p] = cc * V[r][p] - ss * V[r][q]
            V2[r][q] = ss * V[r][p] + cc * V[r][q]
    perm = (0, 2, 3, 1)
    A3 = [[A2[perm[r]][perm[j]] for j in range(4)] for r in range(4)]
    V3 = [[V2[r][perm[j]] for j in range(4)] for r in range(4)]
    return A3, V3


def _eigh3_replica(cxx, cxy, cxz, cyy, cyz, czz):
    zero = jnp.zeros_like(cxx)
    one_v = jnp.ones_like(cxx)
    A = [[cxx, cxy, cxz, zero],
         [cxy, cyy, cyz, zero],
         [cxz, cyz, czz, zero],
         [zero, zero, zero, zero]]
    V = [[one_v, zero, zero, zero],
         [zero, one_v, zero, zero],
         [zero, zero, one_v, zero],
         [zero, zero, zero, one_v]]
    for _ in range(SWEEPS * 3):
        A, V = _jacobi_rotate(A, V)
    w = [A[0][0], A[1][1], A[2][2]]
    cols = [[V[r][j] for r in range(3)] for j in range(3)]

    def cswap(wa, ca, wb, cb):
        pred = wa > wb
        wa2 = jnp.where(pred, wb, wa)
        wb2 = jnp.where(pred, wa, wb)
        ca2 = [jnp.where(pred, yy, xx) for xx, yy in zip(ca, cb)]
        cb2 = [jnp.where(pred, xx, yy) for xx, yy in zip(ca, cb)]
        return wa2, ca2, wb2, cb2

    w[0], cols[0], w[1], cols[1] = cswap(w[0], cols[0], w[1], cols[1])
    w[1], cols[1], w[2], cols[2] = cswap(w[1], cols[1], w[2], cols[2])
    w[0], cols[0], w[1], cols[1] = cswap(w[0], cols[0], w[1], cols[1])
    return w, cols


def _means_kernel(cnt_r, sx_r, sy_r, sz_r, mb_r, mx_r, my_r, mz_r):
    i = pl.program_id(0)
    cnt = cnt_r[...]
    denom = jnp.maximum(cnt, jnp.float32(1.0))
    row = lax.broadcasted_iota(jnp.int32, (BLK_R, BLK_C), 0)
    col = lax.broadcasted_iota(jnp.int32, (BLK_R, BLK_C), 1)
    pos = (i * BLK_R + row) * BLK_C + col
    b0 = (pos // SEG_PER_B).astype(jnp.float32)
    mb_r[...] = jnp.where(cnt > 0, b0, jnp.float32(0.0))
    mx_r[...] = sx_r[...] / denom
    my_r[...] = sy_r[...] / denom
    mz_r[...] = sz_r[...] / denom


def _planes_kernel(cnt_r, mx_r, my_r, mz_r,
                   sxx_r, sxy_r, sxz_r, syy_r, syz_r, szz_r,
                   n0_r, n1_r, n2_r, off_r, e0_r, e1_r, e2_r):
    denom = jnp.maximum(cnt_r[...], jnp.float32(1.0))
    cxx = sxx_r[...] / denom + jnp.float32(1e-4)
    cxy = sxy_r[...] / denom
    cxz = sxz_r[...] / denom
    cyy = syy_r[...] / denom + jnp.float32(2e-4)
    cyz = syz_r[...] / denom
    czz = szz_r[...] / denom + jnp.float32(3e-4)
    w, cols = _eigh3_replica(cxx, cxy, cxz, cyy, cyz, czz)
    n0, n1, n2 = cols[0]
    mx, my, mz = mx_r[...], my_r[...], mz_r[...]
    n0_r[...] = n0
    n1_r[...] = n1
    n2_r[...] = n2
    off_r[...] = -((n0 * mx + n1 * my) + n2 * mz)
    e0_r[...] = w[0]
    e1_r[...] = w[1]
    e2_r[...] = w[2]


def _tc_call(body, n_in, n_out, args):
    spec = pl.BlockSpec((BLK_R, BLK_C), lambda i: (i, 0))
    shape = jax.ShapeDtypeStruct((N_ROWS, BLK_C), jnp.float32)
    return pl.pallas_call(
        body,
        grid=(N_ROWS // BLK_R,),
        in_specs=[spec] * n_in,
        out_specs=[spec] * n_out,
        out_shape=[shape] * n_out,
    )(*args)


def _interleave(channels):
    ncol = len(channels)
    n = channels[0].shape[0]
    col = lax.broadcasted_iota(jnp.int32, (n, ncol), 1)
    out = jnp.zeros((n, ncol), jnp.float32)
    for k, ch in enumerate(channels):
        out = jnp.where(col == k, ch[:, None], out)
    return out


@jax.jit
def kernel(point_bxyz):
    points_lin = point_bxyz.reshape(N_TILES, TILE_PTS, 4)
    points_lin = points_lin.transpose(0, 2, 1).reshape(-1)
    zeros_acc = jnp.zeros((ACC_ROWS,), jnp.float32)
    out4, seg, xyz_soa = _sc_pass1(points_lin, zeros_acc)
    ch4 = out4.reshape(4, N_ROWS, BLK_C)

    mb, mx, my, mz = _tc_call(
        _means_kernel, 4, 4, [ch4[0], ch4[1], ch4[2], ch4[3]])

    def _unpad(a):
        return a.reshape(N_PAD)[:NUM_SEG]

    mean_bxyz = _interleave([_unpad(mb), _unpad(mx), _unpad(my), _unpad(mz)])

    (out6,) = _sc_pass2(xyz_soa, seg, mx.reshape(-1), my.reshape(-1),
                        mz.reshape(-1), zeros_acc)
    ch6 = out6.reshape(6, N_ROWS, BLK_C)

    planes_ch = _tc_call(
        _planes_kernel, 10, 7,
        [ch4[0], mx, my, mz,
         ch6[0], ch6[1], ch6[2], ch6[3], ch6[4], ch6[5]])

    planes = _interleave([_unpad(o) for o in planes_ch])
    return (mean_bxyz, planes)
